# Initial kernel scaffold; baseline (speedup 1.0000x reference)
import jax
import jax.numpy as jnp
from jax.experimental import pallas as pl


def kernel(x, edge_index, edge_attr, Wl0, bl0, Wr0, br0, att0, We0, cb0, lg0, lb0, Wl1, bl1, Wr1, br1, att1, We1, cb1, lg1, lb1, Wl2, bl2, Wr2, br2, att2, We2, cb2, lg2, lb2, Wp, bp):
    raise NotImplementedError("write your pallas kernel here")


# speedup vs baseline: 16.9890x; 16.9890x over previous
import functools

import jax
import jax.numpy as jnp
from jax import lax
from jax.experimental import pallas as pl
from jax.experimental.pallas import tpu as pltpu
from jax.experimental.pallas import tpu_sc as plsc

N_NODES = 10000
DIM = 128
NH = 8
E_EDGES = 320000
E_TOT = E_EDGES + N_NODES
NP = 10240
NW = 32
EW = 10752
EP = NW * EW
CHUNK = 512
NSUB = CHUNK // 128
NCH = EW // CHUNK
ZR = NP // 16
SCH = 128
NCHS = EW // SCH

_MESH = plsc.VectorSubcoreMesh(core_axis_name="c", subcore_axis_name="s")


@functools.partial(
    pl.kernel,
    out_type=jax.ShapeDtypeStruct((EP, DIM), jnp.float32),
    mesh=_MESH,
    scratch_types=[
        pltpu.VMEM((NSUB, 128), jnp.int32),
        pltpu.VMEM((CHUNK, DIM), jnp.float32),
        pltpu.SemaphoreType.DMA,
    ],
)
def _sc_gather128(table_hbm, idx_hbm, out_hbm, idx_v, rows_v, sem):
    wid = lax.axis_index("s") * 2 + lax.axis_index("c")
    for k in range(NCH):
        base = wid * EW + k * CHUNK
        pltpu.sync_copy(idx_hbm.at[pl.ds(wid * (EW // 128) + k * NSUB, NSUB)],
                        idx_v)
        descs = [
            pltpu.async_copy(table_hbm.at[idx_v.at[j]],
                             rows_v.at[pl.ds(j * 128, 128)], sem)
            for j in range(NSUB)
        ]
        for d in descs:
            d.wait()
        pltpu.sync_copy(rows_v, out_hbm.at[pl.ds(base, CHUNK)])


@functools.partial(
    pl.kernel,
    out_type=jax.ShapeDtypeStruct((2 * NP, DIM), jnp.float32),
    mesh=_MESH,
    scratch_types=[
        pltpu.VMEM((1, 128), jnp.int32),
        pltpu.VMEM((SCH, DIM), jnp.float32),
        pltpu.VMEM_SHARED((NP, DIM), jnp.float32),
    ],
)
def _sc_scatter128(rows_hbm, idx_hbm, zeros_hbm, out_hbm, idx_v, rows_v, acc):
    cid = lax.axis_index("c")
    sid = lax.axis_index("s")
    wid = sid * 2 + cid
    pltpu.sync_copy(zeros_hbm.at[pl.ds(sid * ZR, ZR)],
                    acc.at[pl.ds(sid * ZR, ZR)])
    plsc.subcore_barrier()
    for k in range(NCHS):
        base = wid * EW + k * SCH
        pltpu.sync_copy(idx_hbm.at[pl.ds(wid * (EW // 128) + k, 1)], idx_v)
        pltpu.sync_copy(rows_hbm.at[pl.ds(base, SCH)], rows_v)
        pltpu.sync_copy(rows_v, acc.at[idx_v.at[0]], add=True)
    plsc.subcore_barrier()
    pltpu.sync_copy(acc.at[pl.ds(sid * ZR, ZR)],
                    out_hbm.at[pl.ds(cid * NP + sid * ZR, ZR)])


BN = 512
BE = 1024


def _dense_body(h_ref, wl_ref, bl_ref, wr_ref, br_ref, xl_ref, xr_ref):
    hb = h_ref[...]
    xl_ref[...] = jnp.dot(hb, wl_ref[...], preferred_element_type=jnp.float32) + bl_ref[...]
    xr_ref[...] = jnp.dot(hb, wr_ref[...], preferred_element_type=jnp.float32) + br_ref[...]


def _tc_dense(h, wl, bl, wr, br):
    return pl.pallas_call(
        _dense_body,
        grid=(NP // BN,),
        in_specs=[
            pl.BlockSpec((BN, DIM), lambda i: (i, 0)),
            pl.BlockSpec((DIM, DIM), lambda i: (0, 0)),
            pl.BlockSpec((1, DIM), lambda i: (0, 0)),
            pl.BlockSpec((DIM, DIM), lambda i: (0, 0)),
            pl.BlockSpec((1, DIM), lambda i: (0, 0)),
        ],
        out_specs=[pl.BlockSpec((BN, DIM), lambda i: (i, 0))] * 2,
        out_shape=[jax.ShapeDtypeStruct((NP, DIM), jnp.float32)] * 2,
    )(h, wl, bl, wr, br)


def _edge_body(xls_ref, xrd_ref, ea_ref, we_ref, attd_ref, r_ref, s_ref,
               contrib_ref, expad_ref):
    xls = xls_ref[...]
    m = xls + xrd_ref[...] + jnp.dot(
        ea_ref[...], we_ref[...], preferred_element_type=jnp.float32)
    m = jnp.where(m >= 0, m, 0.2 * m)
    ex16 = jnp.exp(jnp.dot(m, attd_ref[...],
                           preferred_element_type=jnp.float32))
    contrib_ref[...] = jnp.dot(
        ex16, r_ref[...], preferred_element_type=jnp.float32) * xls
    expad_ref[...] = jnp.dot(
        ex16, s_ref[...], preferred_element_type=jnp.float32)


def _tc_edge(xls, xrd, ea8, we8, attd16, r16, s16):
    return pl.pallas_call(
        _edge_body,
        grid=(EP // BE,),
        in_specs=[
            pl.BlockSpec((BE, DIM), lambda i: (i, 0)),
            pl.BlockSpec((BE, DIM), lambda i: (i, 0)),
            pl.BlockSpec((BE, 8), lambda i: (i, 0)),
            pl.BlockSpec((8, DIM), lambda i: (0, 0)),
            pl.BlockSpec((DIM, 16), lambda i: (0, 0)),
            pl.BlockSpec((16, DIM), lambda i: (0, 0)),
            pl.BlockSpec((16, DIM), lambda i: (0, 0)),
        ],
        out_specs=[pl.BlockSpec((BE, DIM), lambda i: (i, 0))] * 2,
        out_shape=[jax.ShapeDtypeStruct((EP, DIM), jnp.float32)] * 2,
    )(xls, xrd, ea8, we8, attd16, r16, s16)


def _post01_body(o0_ref, o1_ref, d0_ref, d1_ref, r_ref, cb_ref, lg_ref,
                 lb_ref, h_ref, out_ref):
    d16 = d0_ref[...][:, :16] + d1_ref[...][:, :16]
    db = jnp.dot(d16, r_ref[...], preferred_element_type=jnp.float32)
    o = (o0_ref[...] + o1_ref[...]) / (db + 1e-16) + cb_ref[...]
    o = jnp.where(o > 0, o, jnp.exp(o) - 1.0)
    mu = jnp.mean(o, axis=-1, keepdims=True)
    var = jnp.mean((o - mu) ** 2, axis=-1, keepdims=True)
    z = (o - mu) * lax.rsqrt(var + 1e-5) * lg_ref[...] + lb_ref[...]
    out_ref[...] = z + h_ref[...]


def _tc_post01(o0, o1, d0, d1, r16, cb, lg, lb, h):
    return pl.pallas_call(
        _post01_body,
        grid=(NP // BN,),
        in_specs=[
            pl.BlockSpec((BN, DIM), lambda i: (i, 0)),
            pl.BlockSpec((BN, DIM), lambda i: (i, 0)),
            pl.BlockSpec((BN, DIM), lambda i: (i, 0)),
            pl.BlockSpec((BN, DIM), lambda i: (i, 0)),
            pl.BlockSpec((16, DIM), lambda i: (0, 0)),
            pl.BlockSpec((1, DIM), lambda i: (0, 0)),
            pl.BlockSpec((1, DIM), lambda i: (0, 0)),
            pl.BlockSpec((1, DIM), lambda i: (0, 0)),
            pl.BlockSpec((BN, DIM), lambda i: (i, 0)),
        ],
        out_specs=pl.BlockSpec((BN, DIM), lambda i: (i, 0)),
        out_shape=jax.ShapeDtypeStruct((NP, DIM), jnp.float32),
    )(o0, o1, d0, d1, r16, cb, lg, lb, h)


def _post2_body(o0_ref, o1_ref, d0_ref, d1_ref, r_ref, m16_ref, cb_ref,
                lg_ref, lb_ref, wp_ref, bp_ref, y_ref):
    d16 = d0_ref[...][:, :16] + d1_ref[...][:, :16]
    db = jnp.dot(d16, r_ref[...], preferred_element_type=jnp.float32)
    g = (o0_ref[...] + o1_ref[...]) / (db + 1e-16)
    o16 = jnp.dot(g, m16_ref[...], preferred_element_type=jnp.float32) + cb_ref[...]
    o16 = jnp.where(o16 > 0, o16, jnp.exp(o16) - 1.0)
    mu = jnp.mean(o16, axis=-1, keepdims=True)
    var = jnp.mean((o16 - mu) ** 2, axis=-1, keepdims=True)
    z = (o16 - mu) * lax.rsqrt(var + 1e-5) * lg_ref[...] + lb_ref[...]
    y_ref[...] = jnp.dot(z, wp_ref[...], preferred_element_type=jnp.float32) + bp_ref[...]


def _tc_post2(o0, o1, d0, d1, r16, m16, cb, lg, lb, wp, bp):
    return pl.pallas_call(
        _post2_body,
        grid=(NP // BN,),
        in_specs=[
            pl.BlockSpec((BN, DIM), lambda i: (i, 0)),
            pl.BlockSpec((BN, DIM), lambda i: (i, 0)),
            pl.BlockSpec((BN, DIM), lambda i: (i, 0)),
            pl.BlockSpec((BN, DIM), lambda i: (i, 0)),
            pl.BlockSpec((16, DIM), lambda i: (0, 0)),
            pl.BlockSpec((DIM, 16), lambda i: (0, 0)),
            pl.BlockSpec((1, 16), lambda i: (0, 0)),
            pl.BlockSpec((1, 16), lambda i: (0, 0)),
            pl.BlockSpec((1, 16), lambda i: (0, 0)),
            pl.BlockSpec((16, DIM), lambda i: (0, 0)),
            pl.BlockSpec((1, DIM), lambda i: (0, 0)),
        ],
        out_specs=pl.BlockSpec((BN, DIM), lambda i: (i, 0)),
        out_shape=jax.ShapeDtypeStruct((NP, DIM), jnp.float32),
    )(o0, o1, d0, d1, r16, m16, cb, lg, lb, wp, bp)


def kernel(x, edge_index, edge_attr,
           Wl0, bl0, Wr0, br0, att0, We0, cb0, lg0, lb0,
           Wl1, bl1, Wr1, br1, att1, We1, cb1, lg1, lb1,
           Wl2, bl2, Wr2, br2, att2, We2, cb2, lg2, lb2,
           Wp, bp):
    f32 = jnp.float32
    loop = jnp.arange(N_NODES, dtype=jnp.int32)
    src = jnp.concatenate(
        [edge_index[0], loop, jnp.zeros((EP - E_TOT,), jnp.int32)])
    dst = jnp.concatenate(
        [edge_index[1], loop, jnp.full((EP - E_TOT,), N_NODES, jnp.int32)])
    src2d = src.reshape(EP // 128, 128)
    dst2d = dst.reshape(EP // 128, 128)
    fill = jnp.mean(edge_attr, axis=0, keepdims=True)
    ea8 = jnp.concatenate([
        jnp.pad(edge_attr, ((0, 0), (0, 4))),
        jnp.pad(jnp.tile(fill, (N_NODES, 1)), ((0, 0), (0, 4))),
        jnp.zeros((EP - E_TOT, 8), f32),
    ], axis=0)
    zeros128 = jnp.zeros((NP, DIM), f32)
    r16 = jnp.pad(jnp.repeat(jnp.eye(NH, dtype=f32), 16, axis=1),
                  ((0, 8), (0, 0)))
    s16 = jnp.pad(jnp.eye(NH, dtype=f32), ((0, 8), (0, DIM - NH)))
    m16 = jnp.tile(jnp.eye(16, dtype=f32), (NH, 1)) / NH

    h = jnp.pad(x, ((0, NP - N_NODES), (0, 0)))
    layers = [
        (Wl0, bl0, Wr0, br0, att0, We0, cb0, lg0, lb0),
        (Wl1, bl1, Wr1, br1, att1, We1, cb1, lg1, lb1),
        (Wl2, bl2, Wr2, br2, att2, We2, cb2, lg2, lb2),
    ]
    y = None
    for l, (Wl, bl, Wr, br, att, We, cb, lg, lb) in enumerate(layers):
        we8 = jnp.pad(We, ((0, 4), (0, 0)))
        attd16 = jnp.pad(
            (jnp.eye(NH, dtype=f32)[:, None, :] * att[:, :, None]).reshape(DIM, NH),
            ((0, 0), (0, 8)))
        xl, xr = _tc_dense(h, Wl, bl.reshape(1, DIM), Wr, br.reshape(1, DIM))
        xls = _sc_gather128(xl, src2d)
        xrd = _sc_gather128(xr, dst2d)
        contrib, expad = _tc_edge(xls, xrd, ea8, we8, attd16, r16, s16)
        outp = _sc_scatter128(contrib, dst2d, zeros128)
        denp = _sc_scatter128(expad, dst2d, zeros128)
        if l < 2:
            h = _tc_post01(outp[:NP], outp[NP:], denp[:NP], denp[NP:], r16,
                           cb.reshape(1, DIM), lg.reshape(1, DIM),
                           lb.reshape(1, DIM), h)
        else:
            y = _tc_post2(outp[:NP], outp[NP:], denp[:NP], denp[NP:], r16,
                          m16, cb.reshape(1, 16), lg.reshape(1, 16),
                          lb.reshape(1, 16), Wp, bp.reshape(1, DIM))
    return y[:N_NODES]


# speedup vs baseline: 26.6029x; 1.5659x over previous
import functools

import jax
import jax.numpy as jnp
from jax import lax
from jax.experimental import pallas as pl
from jax.experimental.pallas import tpu as pltpu
from jax.experimental.pallas import tpu_sc as plsc

N_NODES = 10000
DIM = 128
NH = 8
E_EDGES = 320000
E_TOT = E_EDGES + N_NODES
NP = 10240
NW = 32
EW = 10752
EP = NW * EW
CHUNK = 512
NSUB = CHUNK // 128
NCH = EW // CHUNK
ZR = NP // 16
SCH = 128
NCHS = EW // SCH

_MESH = plsc.VectorSubcoreMesh(core_axis_name="c", subcore_axis_name="s")


GCH = 128
NCHG = EW // GCH


@functools.partial(
    pl.kernel,
    out_type=[jax.ShapeDtypeStruct((EP, DIM), jnp.float32)] * 2,
    mesh=_MESH,
    scratch_types=[
        pltpu.VMEM((2, 128), jnp.int32),
        pltpu.VMEM((2, 128), jnp.int32),
        pltpu.VMEM((2, GCH, DIM), jnp.float32),
        pltpu.VMEM((2, GCH, DIM), jnp.float32),
        pltpu.SemaphoreType.DMA,
        pltpu.SemaphoreType.DMA,
        pltpu.SemaphoreType.DMA,
        pltpu.SemaphoreType.DMA,
        pltpu.SemaphoreType.DMA,
        pltpu.SemaphoreType.DMA,
    ],
)
def _sc_gather2(xl_hbm, xr_hbm, sidx_hbm, didx_hbm, xls_out, xrd_out,
                si_v, di_v, ra_v, rb_v, si0, si1, sg0, sg1, sw0, sw1):
    wid = lax.axis_index("s") * 2 + lax.axis_index("c")
    semi = (si0, si1)
    semg = (sg0, sg1)
    semw = (sw0, sw1)
    irow = wid * NCHG

    def load_idx(k):
        b = k & 1
        return [
            pltpu.async_copy(sidx_hbm.at[pl.ds(irow + k, 1)],
                             si_v.at[pl.ds(b, 1)], semi[b]),
            pltpu.async_copy(didx_hbm.at[pl.ds(irow + k, 1)],
                             di_v.at[pl.ds(b, 1)], semi[b]),
        ]

    def start_gather(k):
        b = k & 1
        return [
            pltpu.async_copy(xl_hbm.at[si_v.at[b]], ra_v.at[b], semg[b]),
            pltpu.async_copy(xr_hbm.at[di_v.at[b]], rb_v.at[b], semg[b]),
        ]

    idesc = {0: load_idx(0), 1: load_idx(1)}
    for d in idesc[0]:
        d.wait()
    gdesc = {0: start_gather(0)}
    wdesc = {}
    for k in range(NCHG):
        b = k & 1
        if k + 1 < NCHG:
            for d in idesc[k + 1]:
                d.wait()
            if k - 1 >= 0:
                for d in wdesc[k - 1]:
                    d.wait()
            gdesc[k + 1] = start_gather(k + 1)
        for d in gdesc[k]:
            d.wait()
        base = wid * EW + k * GCH
        wdesc[k] = [
            pltpu.async_copy(ra_v.at[b], xls_out.at[pl.ds(base, GCH)], semw[b]),
            pltpu.async_copy(rb_v.at[b], xrd_out.at[pl.ds(base, GCH)], semw[b]),
        ]
        if k + 2 < NCHG:
            idesc[k + 2] = load_idx(k + 2)
    for d in wdesc[NCHG - 2]:
        d.wait()
    for d in wdesc[NCHG - 1]:
        d.wait()


@functools.partial(
    pl.kernel,
    out_type=jax.ShapeDtypeStruct((2 * NP, DIM), jnp.float32),
    mesh=_MESH,
    scratch_types=[
        pltpu.VMEM((2, 128), jnp.int32),
        pltpu.VMEM((2, SCH, DIM), jnp.float32),
        pltpu.VMEM_SHARED((NP, DIM), jnp.float32),
        pltpu.SemaphoreType.DMA,
        pltpu.SemaphoreType.DMA,
    ],
)
def _sc_scatter128(rows_hbm, idx_hbm, zeros_hbm, out_hbm, idx_v, rows_v, acc,
                   sl0, sl1):
    cid = lax.axis_index("c")
    sid = lax.axis_index("s")
    wid = sid * 2 + cid
    sem = (sl0, sl1)

    def load(k):
        b = k & 1
        return [
            pltpu.async_copy(idx_hbm.at[pl.ds(wid * NCHS + k, 1)],
                             idx_v.at[pl.ds(b, 1)], sem[b]),
            pltpu.async_copy(rows_hbm.at[pl.ds(wid * EW + k * SCH, SCH)],
                             rows_v.at[b], sem[b]),
        ]

    ldesc = {0: load(0)}
    pltpu.sync_copy(zeros_hbm.at[pl.ds(sid * ZR, ZR)],
                    acc.at[pl.ds(sid * ZR, ZR)])
    plsc.subcore_barrier()
    for k in range(NCHS):
        b = k & 1
        if k + 1 < NCHS:
            ldesc[k + 1] = load(k + 1)
        for d in ldesc[k]:
            d.wait()
        pltpu.sync_copy(rows_v.at[b], acc.at[idx_v.at[b]], add=True)
    plsc.subcore_barrier()
    pltpu.sync_copy(acc.at[pl.ds(sid * ZR, ZR)],
                    out_hbm.at[pl.ds(cid * NP + sid * ZR, ZR)])


BN = 512
BE = 1024


def _dense_body(h_ref, wl_ref, bl_ref, wr_ref, br_ref, xl_ref, xr_ref):
    hb = h_ref[...]
    xl_ref[...] = jnp.dot(hb, wl_ref[...], preferred_element_type=jnp.float32) + bl_ref[...]
    xr_ref[...] = jnp.dot(hb, wr_ref[...], preferred_element_type=jnp.float32) + br_ref[...]


def _tc_dense(h, wl, bl, wr, br):
    return pl.pallas_call(
        _dense_body,
        grid=(NP // BN,),
        in_specs=[
            pl.BlockSpec((BN, DIM), lambda i: (i, 0)),
            pl.BlockSpec((DIM, DIM), lambda i: (0, 0)),
            pl.BlockSpec((1, DIM), lambda i: (0, 0)),
            pl.BlockSpec((DIM, DIM), lambda i: (0, 0)),
            pl.BlockSpec((1, DIM), lambda i: (0, 0)),
        ],
        out_specs=[pl.BlockSpec((BN, DIM), lambda i: (i, 0))] * 2,
        out_shape=[jax.ShapeDtypeStruct((NP, DIM), jnp.float32)] * 2,
    )(h, wl, bl, wr, br)


def _edge_body(xls_ref, xrd_ref, ea_ref, we_ref, attd_ref, r_ref, s_ref,
               contrib_ref, expad_ref):
    xls = xls_ref[...]
    m = xls + xrd_ref[...] + jnp.dot(
        ea_ref[...], we_ref[...], preferred_element_type=jnp.float32)
    m = jnp.where(m >= 0, m, 0.2 * m)
    ex16 = jnp.exp(jnp.dot(m, attd_ref[...],
                           preferred_element_type=jnp.float32))
    contrib_ref[...] = jnp.dot(
        ex16, r_ref[...], preferred_element_type=jnp.float32) * xls
    expad_ref[...] = jnp.dot(
        ex16, s_ref[...], preferred_element_type=jnp.float32)


def _tc_edge(xls, xrd, ea8, we8, attd16, r16, s16):
    return pl.pallas_call(
        _edge_body,
        grid=(EP // BE,),
        in_specs=[
            pl.BlockSpec((BE, DIM), lambda i: (i, 0)),
            pl.BlockSpec((BE, DIM), lambda i: (i, 0)),
            pl.BlockSpec((BE, 8), lambda i: (i, 0)),
            pl.BlockSpec((8, DIM), lambda i: (0, 0)),
            pl.BlockSpec((DIM, 16), lambda i: (0, 0)),
            pl.BlockSpec((16, DIM), lambda i: (0, 0)),
            pl.BlockSpec((16, DIM), lambda i: (0, 0)),
        ],
        out_specs=[pl.BlockSpec((BE, DIM), lambda i: (i, 0))] * 2,
        out_shape=[jax.ShapeDtypeStruct((EP, DIM), jnp.float32)] * 2,
    )(xls, xrd, ea8, we8, attd16, r16, s16)


def _post01_body(o0_ref, o1_ref, d0_ref, d1_ref, r_ref, cb_ref, lg_ref,
                 lb_ref, h_ref, out_ref):
    d16 = d0_ref[...][:, :16] + d1_ref[...][:, :16]
    db = jnp.dot(d16, r_ref[...], preferred_element_type=jnp.float32)
    o = (o0_ref[...] + o1_ref[...]) / (db + 1e-16) + cb_ref[...]
    o = jnp.where(o > 0, o, jnp.exp(o) - 1.0)
    mu = jnp.mean(o, axis=-1, keepdims=True)
    var = jnp.mean((o - mu) ** 2, axis=-1, keepdims=True)
    z = (o - mu) * lax.rsqrt(var + 1e-5) * lg_ref[...] + lb_ref[...]
    out_ref[...] = z + h_ref[...]


def _tc_post01(o0, o1, d0, d1, r16, cb, lg, lb, h):
    return pl.pallas_call(
        _post01_body,
        grid=(NP // BN,),
        in_specs=[
            pl.BlockSpec((BN, DIM), lambda i: (i, 0)),
            pl.BlockSpec((BN, DIM), lambda i: (i, 0)),
            pl.BlockSpec((BN, DIM), lambda i: (i, 0)),
            pl.BlockSpec((BN, DIM), lambda i: (i, 0)),
            pl.BlockSpec((16, DIM), lambda i: (0, 0)),
            pl.BlockSpec((1, DIM), lambda i: (0, 0)),
            pl.BlockSpec((1, DIM), lambda i: (0, 0)),
            pl.BlockSpec((1, DIM), lambda i: (0, 0)),
            pl.BlockSpec((BN, DIM), lambda i: (i, 0)),
        ],
        out_specs=pl.BlockSpec((BN, DIM), lambda i: (i, 0)),
        out_shape=jax.ShapeDtypeStruct((NP, DIM), jnp.float32),
    )(o0, o1, d0, d1, r16, cb, lg, lb, h)


def _post2_body(o0_ref, o1_ref, d0_ref, d1_ref, r_ref, m16_ref, cb_ref,
                lg_ref, lb_ref, wp_ref, bp_ref, y_ref):
    d16 = d0_ref[...][:, :16] + d1_ref[...][:, :16]
    db = jnp.dot(d16, r_ref[...], preferred_element_type=jnp.float32)
    g = (o0_ref[...] + o1_ref[...]) / (db + 1e-16)
    o16 = jnp.dot(g, m16_ref[...], preferred_element_type=jnp.float32) + cb_ref[...]
    o16 = jnp.where(o16 > 0, o16, jnp.exp(o16) - 1.0)
    mu = jnp.mean(o16, axis=-1, keepdims=True)
    var = jnp.mean((o16 - mu) ** 2, axis=-1, keepdims=True)
    z = (o16 - mu) * lax.rsqrt(var + 1e-5) * lg_ref[...] + lb_ref[...]
    y_ref[...] = jnp.dot(z, wp_ref[...], preferred_element_type=jnp.float32) + bp_ref[...]


def _tc_post2(o0, o1, d0, d1, r16, m16, cb, lg, lb, wp, bp):
    return pl.pallas_call(
        _post2_body,
        grid=(NP // BN,),
        in_specs=[
            pl.BlockSpec((BN, DIM), lambda i: (i, 0)),
            pl.BlockSpec((BN, DIM), lambda i: (i, 0)),
            pl.BlockSpec((BN, DIM), lambda i: (i, 0)),
            pl.BlockSpec((BN, DIM), lambda i: (i, 0)),
            pl.BlockSpec((16, DIM), lambda i: (0, 0)),
            pl.BlockSpec((DIM, 16), lambda i: (0, 0)),
            pl.BlockSpec((1, 16), lambda i: (0, 0)),
            pl.BlockSpec((1, 16), lambda i: (0, 0)),
            pl.BlockSpec((1, 16), lambda i: (0, 0)),
            pl.BlockSpec((16, DIM), lambda i: (0, 0)),
            pl.BlockSpec((1, DIM), lambda i: (0, 0)),
        ],
        out_specs=pl.BlockSpec((BN, DIM), lambda i: (i, 0)),
        out_shape=jax.ShapeDtypeStruct((NP, DIM), jnp.float32),
    )(o0, o1, d0, d1, r16, m16, cb, lg, lb, wp, bp)


def kernel(x, edge_index, edge_attr,
           Wl0, bl0, Wr0, br0, att0, We0, cb0, lg0, lb0,
           Wl1, bl1, Wr1, br1, att1, We1, cb1, lg1, lb1,
           Wl2, bl2, Wr2, br2, att2, We2, cb2, lg2, lb2,
           Wp, bp):
    f32 = jnp.float32
    loop = jnp.arange(N_NODES, dtype=jnp.int32)
    src = jnp.concatenate(
        [edge_index[0], loop, jnp.zeros((EP - E_TOT,), jnp.int32)])
    dst = jnp.concatenate(
        [edge_index[1], loop, jnp.full((EP - E_TOT,), N_NODES, jnp.int32)])
    src2d = src.reshape(EP // 128, 128)
    dst2d = dst.reshape(EP // 128, 128)
    fill = jnp.mean(edge_attr, axis=0, keepdims=True)
    ea8 = jnp.concatenate([
        jnp.pad(edge_attr, ((0, 0), (0, 4))),
        jnp.pad(jnp.tile(fill, (N_NODES, 1)), ((0, 0), (0, 4))),
        jnp.zeros((EP - E_TOT, 8), f32),
    ], axis=0)
    zeros128 = jnp.zeros((NP, DIM), f32)
    r16 = jnp.pad(jnp.repeat(jnp.eye(NH, dtype=f32), 16, axis=1),
                  ((0, 8), (0, 0)))
    s16 = jnp.pad(jnp.eye(NH, dtype=f32), ((0, 8), (0, DIM - NH)))
    m16 = jnp.tile(jnp.eye(16, dtype=f32), (NH, 1)) / NH

    h = jnp.pad(x, ((0, NP - N_NODES), (0, 0)))
    layers = [
        (Wl0, bl0, Wr0, br0, att0, We0, cb0, lg0, lb0),
        (Wl1, bl1, Wr1, br1, att1, We1, cb1, lg1, lb1),
        (Wl2, bl2, Wr2, br2, att2, We2, cb2, lg2, lb2),
    ]
    y = None
    for l, (Wl, bl, Wr, br, att, We, cb, lg, lb) in enumerate(layers):
        we8 = jnp.pad(We, ((0, 4), (0, 0)))
        attd16 = jnp.pad(
            (jnp.eye(NH, dtype=f32)[:, None, :] * att[:, :, None]).reshape(DIM, NH),
            ((0, 0), (0, 8)))
        xl, xr = _tc_dense(h, Wl, bl.reshape(1, DIM), Wr, br.reshape(1, DIM))
        xls, xrd = _sc_gather2(xl, xr, src2d, dst2d)
        contrib, expad = _tc_edge(xls, xrd, ea8, we8, attd16, r16, s16)
        outp = _sc_scatter128(contrib, dst2d, zeros128)
        denp = _sc_scatter128(expad, dst2d, zeros128)
        if l < 2:
            h = _tc_post01(outp[:NP], outp[NP:], denp[:NP], denp[NP:], r16,
                           cb.reshape(1, DIM), lg.reshape(1, DIM),
                           lb.reshape(1, DIM), h)
        else:
            y = _tc_post2(outp[:NP], outp[NP:], denp[:NP], denp[NP:], r16,
                          m16, cb.reshape(1, 16), lg.reshape(1, 16),
                          lb.reshape(1, 16), Wp, bp.reshape(1, DIM))
    return y[:N_NODES]
